# baseline (speedup 1.0000x reference)
import functools

import jax
import jax.numpy as jnp
from jax import lax
from jax.experimental import pallas as pl
from jax.experimental.pallas import tpu as pltpu
from jax.experimental.pallas import tpu_sc as plsc

VOCAB = 1000
EMBD = 32
BATCH = 1024
SEQ = 50
N = BATCH * SEQ

NUM_SC = 2
NUM_SUBCORES = 16
NW = NUM_SC * NUM_SUBCORES
ROWS_PER_W = N // NW
CHUNK = 80
N_CHUNKS = ROWS_PER_W // CHUNK

BM = 800


def _sc_gather(table, idx_flat):
  mesh = plsc.VectorSubcoreMesh(core_axis_name="c", subcore_axis_name="s")

  @functools.partial(
      pl.kernel,
      out_type=jax.ShapeDtypeStruct((N, EMBD), jnp.float32),
      mesh=mesh,
      scratch_types=[
          pltpu.VMEM((ROWS_PER_W,), jnp.int32),
          pltpu.VMEM((ROWS_PER_W, EMBD), jnp.float32),
          pltpu.SemaphoreType.DMA,
      ],
      compiler_params=pltpu.CompilerParams(use_tc_tiling_on_sc=False),
  )
  def gather_kernel(table_hbm, idx_hbm, out_hbm, idx_v, rows_v, sem):
    wid = lax.axis_index("s") * NUM_SC + lax.axis_index("c")
    base = wid * ROWS_PER_W
    pltpu.sync_copy(idx_hbm.at[pl.ds(base, ROWS_PER_W)], idx_v)
    copies = [
        pltpu.async_copy(
            table_hbm.at[idx_v.at[pl.ds(j * CHUNK, CHUNK)]],
            rows_v.at[pl.ds(j * CHUNK, CHUNK)],
            sem,
        )
        for j in range(N_CHUNKS)
    ]
    for c in copies:
      c.wait()
    pltpu.sync_copy(rows_v, out_hbm.at[pl.ds(base, ROWS_PER_W)])

  return gather_kernel(table, idx_flat)


def _tc_head(emb, pos_tiled, w, bias):
  def head(emb_ref, pos_ref, w_ref, b_ref, out_ref):
    x = emb_ref[...] + pos_ref[...]
    out_ref[...] = (
        jnp.dot(x, w_ref[...], preferred_element_type=jnp.float32)
        + b_ref[...]
    )

  return pl.pallas_call(
      head,
      grid=(N // BM,),
      in_specs=[
          pl.BlockSpec((BM, EMBD), lambda i: (i, 0)),
          pl.BlockSpec((BM, EMBD), lambda i: (0, 0)),
          pl.BlockSpec((EMBD, VOCAB), lambda i: (0, 0)),
          pl.BlockSpec((1, VOCAB), lambda i: (0, 0)),
      ],
      out_specs=pl.BlockSpec((BM, VOCAB), lambda i: (i, 0)),
      out_shape=jax.ShapeDtypeStruct((N, VOCAB), jnp.float32),
      compiler_params=pltpu.CompilerParams(
          dimension_semantics=("arbitrary",),
      ),
  )(emb, pos_tiled, w, bias)


def kernel(idx, tok_table, pos_table, W, b):
  idx_flat = idx.reshape(N).astype(jnp.int32)
  emb = _sc_gather(tok_table, idx_flat)
  pos_tiled = jnp.tile(pos_table, (BM // SEQ, 1))
  logits = _tc_head(emb, pos_tiled, W, b.reshape(1, VOCAB))
  return logits.reshape(BATCH, SEQ, VOCAB)


# speedup vs baseline: 1.0066x; 1.0066x over previous
import functools

import jax
import jax.numpy as jnp
from jax import lax
from jax.experimental import pallas as pl
from jax.experimental.pallas import tpu as pltpu
from jax.experimental.pallas import tpu_sc as plsc

VOCAB = 1000
EMBD = 32
DPAD = 128
BATCH = 1024
SEQ = 50
N = BATCH * SEQ

NUM_SC = 2
NUM_SUBCORES = 16
NW = NUM_SC * NUM_SUBCORES
ROWS_PER_W = N // NW
HALF = ROWS_PER_W // 2
CHUNK = 80
N_CHUNKS = HALF // CHUNK

BM = 800


def _sc_gather(table_pad, idx_flat):
  mesh = plsc.VectorSubcoreMesh(core_axis_name="c", subcore_axis_name="s")

  @functools.partial(
      pl.kernel,
      out_type=jax.ShapeDtypeStruct((N, DPAD), jnp.float32),
      mesh=mesh,
      scratch_types=[
          pltpu.VMEM((ROWS_PER_W,), jnp.int32),
          pltpu.VMEM((HALF, DPAD), jnp.float32),
          pltpu.SemaphoreType.DMA,
      ],
  )
  def gather_kernel(table_hbm, idx_hbm, out_hbm, idx_v, rows_v, sem):
    wid = lax.axis_index("s") * NUM_SC + lax.axis_index("c")
    base = wid * ROWS_PER_W
    pltpu.sync_copy(idx_hbm.at[pl.ds(base, ROWS_PER_W)], idx_v)
    for h in range(2):
      copies = [
          pltpu.async_copy(
              table_hbm.at[idx_v.at[pl.ds(h * HALF + j * CHUNK, CHUNK)]],
              rows_v.at[pl.ds(j * CHUNK, CHUNK)],
              sem,
          )
          for j in range(N_CHUNKS)
      ]
      for c in copies:
        c.wait()
      pltpu.sync_copy(rows_v, out_hbm.at[pl.ds(base + h * HALF, HALF)])

  return gather_kernel(table_pad, idx_flat)


def _tc_head(emb, pos_tiled, w, bias):
  def head(emb_ref, pos_ref, w_ref, b_ref, out_ref):
    x = emb_ref[...] + pos_ref[...]
    out_ref[...] = (
        jnp.dot(x, w_ref[...], preferred_element_type=jnp.float32)
        + b_ref[...]
    )

  return pl.pallas_call(
      head,
      grid=(N // BM,),
      in_specs=[
          pl.BlockSpec((BM, DPAD), lambda i: (i, 0)),
          pl.BlockSpec((BM, DPAD), lambda i: (0, 0)),
          pl.BlockSpec((DPAD, VOCAB), lambda i: (0, 0)),
          pl.BlockSpec((1, VOCAB), lambda i: (0, 0)),
      ],
      out_specs=pl.BlockSpec((BM, VOCAB), lambda i: (i, 0)),
      out_shape=jax.ShapeDtypeStruct((N, VOCAB), jnp.float32),
      compiler_params=pltpu.CompilerParams(
          dimension_semantics=("arbitrary",),
      ),
  )(emb, pos_tiled, w, bias)


def kernel(idx, tok_table, pos_table, W, b):
  idx_flat = idx.reshape(N).astype(jnp.int32)
  tok_pad = jnp.pad(tok_table, ((0, 0), (0, DPAD - EMBD)))
  emb = _sc_gather(tok_pad, idx_flat)
  pos_tiled = jnp.tile(jnp.pad(pos_table, ((0, 0), (0, DPAD - EMBD))),
                       (BM // SEQ, 1))
  w_pad = jnp.pad(W, ((0, DPAD - EMBD), (0, 0)))
  logits = _tc_head(emb, pos_tiled, w_pad, b.reshape(1, VOCAB))
  return logits.reshape(BATCH, SEQ, VOCAB)


# speedup vs baseline: 1.2557x; 1.2475x over previous
import functools

import jax
import jax.numpy as jnp
from jax import lax
from jax.experimental import pallas as pl
from jax.experimental.pallas import tpu as pltpu
from jax.experimental.pallas import tpu_sc as plsc

VOCAB = 1000
EMBD = 32
DPAD = 128
BATCH = 1024
SEQ = 50

NUM_SC = 2
NUM_SUBCORES = 16
NW = NUM_SC * NUM_SUBCORES
B_PER_W = BATCH // NW
B_HALF = B_PER_W // 2

BB = 16


def _sc_gather(table_pad, idx):
  mesh = plsc.VectorSubcoreMesh(core_axis_name="c", subcore_axis_name="s")

  @functools.partial(
      pl.kernel,
      out_type=jax.ShapeDtypeStruct((BATCH, SEQ, DPAD), jnp.float32),
      mesh=mesh,
      scratch_types=[
          pltpu.VMEM((B_PER_W, SEQ), jnp.int32),
          pltpu.VMEM((B_HALF, SEQ, DPAD), jnp.float32),
          pltpu.SemaphoreType.DMA,
      ],
  )
  def gather_kernel(table_hbm, idx_hbm, out_hbm, idx_v, rows_v, sem):
    wid = lax.axis_index("s") * NUM_SC + lax.axis_index("c")
    base = wid * B_PER_W
    pltpu.sync_copy(idx_hbm.at[pl.ds(base, B_PER_W)], idx_v)
    for h in range(2):
      copies = [
          pltpu.async_copy(
              table_hbm.at[idx_v.at[h * B_HALF + j]],
              rows_v.at[j],
              sem,
          )
          for j in range(B_HALF)
      ]
      for c in copies:
        c.wait()
      pltpu.sync_copy(rows_v, out_hbm.at[pl.ds(base + h * B_HALF, B_HALF)])

  return gather_kernel(table_pad, idx)


def _tc_head(emb, pos_pad, w, bias):
  def head(emb_ref, pos_ref, w_ref, b_ref, out_ref):
    pos = pos_ref[...]
    w_v = w_ref[...]
    b_v = b_ref[...]
    for j in range(BB):
      x = emb_ref[j] + pos
      out_ref[j] = (
          jnp.dot(x, w_v, preferred_element_type=jnp.float32) + b_v
      )

  return pl.pallas_call(
      head,
      grid=(BATCH // BB,),
      in_specs=[
          pl.BlockSpec((BB, SEQ, DPAD), lambda i: (i, 0, 0)),
          pl.BlockSpec((SEQ, DPAD), lambda i: (0, 0)),
          pl.BlockSpec((DPAD, VOCAB), lambda i: (0, 0)),
          pl.BlockSpec((1, VOCAB), lambda i: (0, 0)),
      ],
      out_specs=pl.BlockSpec((BB, SEQ, VOCAB), lambda i: (i, 0, 0)),
      out_shape=jax.ShapeDtypeStruct((BATCH, SEQ, VOCAB), jnp.float32),
      compiler_params=pltpu.CompilerParams(
          dimension_semantics=("arbitrary",),
      ),
  )(emb, pos_pad, w, bias)


def kernel(idx, tok_table, pos_table, W, b):
  idx32 = idx.astype(jnp.int32)
  tok_pad = jnp.pad(tok_table, ((0, 0), (0, DPAD - EMBD)))
  emb = _sc_gather(tok_pad, idx32)
  pos_pad = jnp.pad(pos_table, ((0, 0), (0, DPAD - EMBD)))
  w_pad = jnp.pad(W, ((0, DPAD - EMBD), (0, 0)))
  return _tc_head(emb, pos_pad, w_pad, b.reshape(1, VOCAB))
